# baseline (speedup 1.0000x reference)
import jax
import jax.numpy as jnp
from jax import lax
from jax.experimental import pallas as pl
from jax.experimental.pallas import tpu as pltpu
from jax.experimental.pallas import tpu_sc as plsc

B = 4096
L = 200
LP = 208
EMB = 64
VOCAB = 1000000
NCLASS = 5
LANE_PAD = 128

NC = 2
NS = 16
NW = NC * NS
BPW = B // NW
WPW = BPW * LP

TBLW = 128
CB = 8192

SPLIT0 = 128
SPLIT1 = LP - SPLIT0


def _tp_body(in_ref, out_ref):
    x = in_ref[...]
    xt = jnp.transpose(x)
    z = jnp.zeros((CB, TBLW - EMB), jnp.float32)
    out_ref[...] = jnp.concatenate([xt, z], axis=1)


def _transpose_pack(table_t):
    grid = (VOCAB + CB - 1) // CB
    return pl.pallas_call(
        _tp_body,
        grid=(grid,),
        in_specs=[pl.BlockSpec((EMB, CB), lambda j: (0, j))],
        out_specs=pl.BlockSpec((CB, TBLW), lambda j: (j, 0)),
        out_shape=jax.ShapeDtypeStruct((VOCAB, TBLW), jnp.float32),
    )(table_t)


def _sc_body(dataf_hbm, tbl_hbm, outf_hbm, idxf_v, rows_a, rows_b, outf_v,
             sem_a, sem_b):
    wid = lax.axis_index("s") * NC + lax.axis_index("c")

    pltpu.sync_copy(dataf_hbm.at[pl.ds(wid * WPW, WPW)], idxf_v)

    def start(i, rows, sem):
        base = i * LP
        pltpu.async_copy(tbl_hbm.at[idxf_v.at[pl.ds(base, SPLIT0)]],
                         rows.at[pl.ds(0, SPLIT0), :], sem)
        pltpu.async_copy(tbl_hbm.at[idxf_v.at[pl.ds(base + SPLIT0, SPLIT1)]],
                         rows.at[pl.ds(SPLIT0, SPLIT1), :], sem)

    def wait(i, rows, sem):
        base = i * LP
        pltpu.make_async_copy(tbl_hbm.at[idxf_v.at[pl.ds(base, SPLIT0)]],
                              rows.at[pl.ds(0, SPLIT0), :], sem).wait()
        pltpu.make_async_copy(
            tbl_hbm.at[idxf_v.at[pl.ds(base + SPLIT0, SPLIT1)]],
            rows.at[pl.ds(SPLIT0, SPLIT1), :], sem).wait()

    def accum_bag(i, rows):
        def rbody(t, accs):
            accs = list(accs)
            rb = pl.multiple_of(t * 16, 16)
            for u in range(16):
                for c in range(4):
                    v = rows[rb + u, pl.ds(c * 16, 16)]
                    k = c * 2 + (u & 1)
                    accs[k] = accs[k] + v
            return tuple(accs)

        z = jnp.zeros((16,), jnp.float32)
        accs = lax.fori_loop(0, LP // 16, rbody, (z,) * 8)
        ob = pl.multiple_of(i * EMB, 16)
        for k in range(4):
            outf_v[pl.ds(ob + k * 16, 16)] = accs[k * 2] + accs[k * 2 + 1]

    start(0, rows_a, sem_a)

    def body(j, carry):
        i = j * 2
        start(i + 1, rows_b, sem_b)
        wait(i, rows_a, sem_a)
        accum_bag(i, rows_a)

        @pl.when(i + 2 < BPW)
        def _():
            start(i + 2, rows_a, sem_a)

        wait(i + 1, rows_b, sem_b)
        accum_bag(i + 1, rows_b)
        return carry

    lax.fori_loop(0, BPW // 2, body, 0)

    pltpu.sync_copy(outf_v, outf_hbm.at[pl.ds(wid * BPW * EMB, BPW * EMB)])


def _sc_bag_sum(dataf, tbl):
    mesh = plsc.VectorSubcoreMesh(core_axis_name="c", subcore_axis_name="s",
                                  num_cores=NC, num_subcores=NS)
    return pl.kernel(
        _sc_body,
        out_type=jax.ShapeDtypeStruct((B * EMB,), jnp.float32),
        mesh=mesh,
        compiler_params=pltpu.CompilerParams(use_tc_tiling_on_sc=False),
        scratch_types=[
            pltpu.VMEM((WPW,), jnp.int32),
            pltpu.VMEM((LP, TBLW), jnp.float32),
            pltpu.VMEM((LP, TBLW), jnp.float32),
            pltpu.VMEM((BPW * EMB,), jnp.float32),
            pltpu.SemaphoreType.DMA,
            pltpu.SemaphoreType.DMA,
        ],
    )(dataf, tbl)


def _tc_body(sums_ref, data_ref, len_ref, t0_ref, wp_ref, bp_ref, out_ref):
    n0 = jnp.sum((data_ref[...] == 0).astype(jnp.float32), axis=1,
                 keepdims=True) + float(LP - L)
    pooled = (sums_ref[...] - n0 * t0_ref[...]) / len_ref[...].astype(
        jnp.float32)
    logits = jnp.dot(pooled, wp_ref[...],
                     preferred_element_type=jnp.float32) + bp_ref[...]
    m = jnp.max(logits, axis=-1, keepdims=True)
    e = jnp.exp(logits - m)
    s = jnp.sum(e, axis=-1, keepdims=True)
    out_full = logits - m - jnp.log(s)
    out_ref[...] = out_full[:, :NCLASS]


def kernel(data, length, table, W, b):
    data = data.astype(jnp.int32)
    dataf = jnp.pad(data, ((0, 0), (0, LP - L))).reshape(B * LP)
    tbl = _transpose_pack(table.T)
    sums = _sc_bag_sum(dataf, tbl).reshape(B, EMB)

    wp = jnp.zeros((EMB, LANE_PAD), jnp.float32).at[:, :NCLASS].set(W.T)
    bp = jnp.full((1, LANE_PAD), -1e30, jnp.float32).at[0, :NCLASS].set(b)
    t0 = tbl[0:1, :EMB]
    len2 = length.astype(jnp.int32).reshape(B, 1)

    out = pl.pallas_call(
        _tc_body,
        out_shape=jax.ShapeDtypeStruct((B, NCLASS), jnp.float32),
    )(sums, data, len2, t0, wp, bp)
    return out


# speedup vs baseline: 1.6795x; 1.6795x over previous
import jax
import jax.numpy as jnp
from jax import lax
from jax.experimental import pallas as pl
from jax.experimental.pallas import tpu as pltpu
from jax.experimental.pallas import tpu_sc as plsc

B = 4096
L = 200
LP = 208
EMB = 64
VOCAB = 1000000
NCLASS = 5
LANE_PAD = 128

NC = 2
NS = 16
NW = NC * NS
BPW = B // NW
WPW = BPW * LP

TBLW = EMB
CB = 8192

SPLIT0 = 128
SPLIT1 = LP - SPLIT0


def _tp_body(in_ref, out_ref):
    x = in_ref[...]
    xt = jnp.transpose(x)
    z = jnp.zeros((CB, EMB), jnp.float32)
    out_ref[...] = jnp.concatenate([xt, z], axis=1)


def _transpose_pack(table_t):
    return pl.pallas_call(
        _tp_body,
        grid=((VOCAB + CB - 1) // CB,),
        in_specs=[pl.BlockSpec((EMB, CB), lambda j: (0, j))],
        out_specs=pl.BlockSpec((CB, 2 * EMB), lambda j: (j, 0)),
        out_shape=jax.ShapeDtypeStruct((VOCAB, 2 * EMB), jnp.float32),
    )(table_t)


def _sc_body(dataf_hbm, tbl_hbm, outf_hbm, idxf_v, idx2_v, rows_a, rows_b,
             outf_v, sem_a, sem_b):
    wid = lax.axis_index("s") * NC + lax.axis_index("c")

    pltpu.sync_copy(dataf_hbm.at[pl.ds(wid * WPW, WPW)], idxf_v)

    def remap_body(t, carry):
        o = pl.multiple_of(t * 16, 16)
        v = idxf_v[pl.ds(o, 16)]
        idx2_v[pl.ds(o, 16)] = v + v
        return carry

    lax.fori_loop(0, WPW // 16, remap_body, 0)

    def start(i, rows, sem):
        base = i * LP
        pltpu.async_copy(tbl_hbm.at[idx2_v.at[pl.ds(base, SPLIT0)]],
                         rows.at[pl.ds(0, SPLIT0), :], sem)
        pltpu.async_copy(tbl_hbm.at[idx2_v.at[pl.ds(base + SPLIT0, SPLIT1)]],
                         rows.at[pl.ds(SPLIT0, SPLIT1), :], sem)

    def wait(i, rows, sem):
        base = i * LP
        pltpu.make_async_copy(tbl_hbm.at[idx2_v.at[pl.ds(base, SPLIT0)]],
                              rows.at[pl.ds(0, SPLIT0), :], sem).wait()
        pltpu.make_async_copy(
            tbl_hbm.at[idx2_v.at[pl.ds(base + SPLIT0, SPLIT1)]],
            rows.at[pl.ds(SPLIT0, SPLIT1), :], sem).wait()

    def accum_bag(i, rows):
        def rbody(t, accs):
            accs = list(accs)
            rb = pl.multiple_of(t * 16, 16)
            for u in range(16):
                for c in range(4):
                    v = rows[rb + u, pl.ds(c * 16, 16)]
                    k = c * 2 + (u & 1)
                    accs[k] = accs[k] + v
            return tuple(accs)

        z = jnp.zeros((16,), jnp.float32)
        accs = lax.fori_loop(0, LP // 16, rbody, (z,) * 8)
        ob = pl.multiple_of(i * EMB, 16)
        for k in range(4):
            outf_v[pl.ds(ob + k * 16, 16)] = accs[k * 2] + accs[k * 2 + 1]

    start(0, rows_a, sem_a)

    def body(j, carry):
        i = j * 2
        start(i + 1, rows_b, sem_b)
        wait(i, rows_a, sem_a)
        accum_bag(i, rows_a)

        @pl.when(i + 2 < BPW)
        def _():
            start(i + 2, rows_a, sem_a)

        wait(i + 1, rows_b, sem_b)
        accum_bag(i + 1, rows_b)
        return carry

    lax.fori_loop(0, BPW // 2, body, 0)

    pltpu.sync_copy(outf_v, outf_hbm.at[pl.ds(wid * BPW * EMB, BPW * EMB)])


def _sc_bag_sum(dataf, tbl):
    mesh = plsc.VectorSubcoreMesh(core_axis_name="c", subcore_axis_name="s",
                                  num_cores=NC, num_subcores=NS)
    return pl.kernel(
        _sc_body,
        out_type=jax.ShapeDtypeStruct((B * EMB,), jnp.float32),
        mesh=mesh,
        compiler_params=pltpu.CompilerParams(use_tc_tiling_on_sc=False),
        scratch_types=[
            pltpu.VMEM((WPW,), jnp.int32),
            pltpu.VMEM((WPW,), jnp.int32),
            pltpu.VMEM((LP, EMB), jnp.float32),
            pltpu.VMEM((LP, EMB), jnp.float32),
            pltpu.VMEM((BPW * EMB,), jnp.float32),
            pltpu.SemaphoreType.DMA,
            pltpu.SemaphoreType.DMA,
        ],
    )(dataf, tbl)


def _tc_body(sums_ref, data_ref, len_ref, t0_ref, wp_ref, bp_ref, out_ref):
    n0 = jnp.sum((data_ref[...] == 0).astype(jnp.float32), axis=1,
                 keepdims=True) + float(LP - L)
    pooled = (sums_ref[...] - n0 * t0_ref[...]) / len_ref[...].astype(
        jnp.float32)
    logits = jnp.dot(pooled, wp_ref[...],
                     preferred_element_type=jnp.float32) + bp_ref[...]
    m = jnp.max(logits, axis=-1, keepdims=True)
    e = jnp.exp(logits - m)
    s = jnp.sum(e, axis=-1, keepdims=True)
    out_full = logits - m - jnp.log(s)
    out_ref[...] = out_full[:, :NCLASS]


def kernel(data, length, table, W, b):
    data = data.astype(jnp.int32)
    dataf = jnp.pad(data, ((0, 0), (0, LP - L))).reshape(B * LP)
    tbl = _transpose_pack(table.T).reshape(2 * VOCAB, EMB)
    sums = _sc_bag_sum(dataf, tbl).reshape(B, EMB)

    wp = jnp.zeros((EMB, LANE_PAD), jnp.float32).at[:, :NCLASS].set(W.T)
    bp = jnp.full((1, LANE_PAD), -1e30, jnp.float32).at[0, :NCLASS].set(b)
    t0 = tbl[0:1, :]
    len2 = length.astype(jnp.int32).reshape(B, 1)

    out = pl.pallas_call(
        _tc_body,
        out_shape=jax.ShapeDtypeStruct((B, NCLASS), jnp.float32),
    )(sums, data, len2, t0, wp, bp)
    return out


# speedup vs baseline: 4.2205x; 2.5130x over previous
import jax
import jax.numpy as jnp
from jax import lax
from jax.experimental import pallas as pl
from jax.experimental.pallas import tpu as pltpu
from jax.experimental.pallas import tpu_sc as plsc

B = 4096
L = 200
LP = 208
EMB = 64
VOCAB = 1000000
NCLASS = 5
LANE_PAD = 128

NC = 2
NS = 16
NW = NC * NS
BPW = B // NW
WPW = BPW * LP

TBLW = EMB
CB = 8192

SPLIT0 = 128
SPLIT1 = L - SPLIT0


def _tp_body(in_ref, out_ref):
    x = in_ref[...]
    xt = jnp.transpose(x)
    z = jnp.zeros((CB, EMB), jnp.float32)
    out_ref[...] = jnp.concatenate([xt, z], axis=1)


def _transpose_pack(table_t):
    return pl.pallas_call(
        _tp_body,
        grid=((VOCAB + CB - 1) // CB,),
        in_specs=[pl.BlockSpec((EMB, CB), lambda j: (0, j))],
        out_specs=pl.BlockSpec((CB, 2 * EMB), lambda j: (j, 0)),
        out_shape=jax.ShapeDtypeStruct((VOCAB, 2 * EMB), jnp.float32),
    )(table_t)


def _sc_body(data_hbm, tbl_hbm, out_hbm, idx_v, idx2_v, rows_a, rows_b,
             out_v, sem_a, sem_b):
    wid = lax.axis_index("s") * NC + lax.axis_index("c")
    base = wid * BPW

    pltpu.sync_copy(data_hbm.at[pl.ds(base, BPW), :], idx_v)

    def remap_body(i, carry):
        for c in range(12):
            o = c * 16
            idx2_v[i, pl.ds(o, 16)] = idx_v[i, pl.ds(o, 16)] * 2
        idx2_v[i, pl.ds(L - 16, 16)] = idx_v[i, pl.ds(L - 16, 16)] * 2
        return carry

    lax.fori_loop(0, BPW, remap_body, 0)

    def start(i, rows, sem):
        pltpu.async_copy(tbl_hbm.at[idx2_v.at[i, pl.ds(0, SPLIT0)]],
                         rows.at[pl.ds(0, SPLIT0), :], sem)
        pltpu.async_copy(tbl_hbm.at[idx2_v.at[i, pl.ds(SPLIT0, SPLIT1)]],
                         rows.at[pl.ds(SPLIT0, SPLIT1), :], sem)

    def wait(i, rows, sem):
        pltpu.make_async_copy(tbl_hbm.at[idx2_v.at[i, pl.ds(0, SPLIT0)]],
                              rows.at[pl.ds(0, SPLIT0), :], sem).wait()
        pltpu.make_async_copy(tbl_hbm.at[idx2_v.at[i, pl.ds(SPLIT0, SPLIT1)]],
                              rows.at[pl.ds(SPLIT0, SPLIT1), :], sem).wait()

    def accum_bag(i, rows):
        def rbody(r, accs):
            accs = list(accs)
            rb = r * 8
            for u in range(8):
                for c in range(4):
                    v = rows[rb + u, pl.ds(c * 16, 16)]
                    k = c * 2 + (u & 1)
                    accs[k] = accs[k] + v
            return tuple(accs)

        z = jnp.zeros((16,), jnp.float32)
        accs = lax.fori_loop(0, L // 8, rbody, (z,) * 8)
        for c in range(4):
            out_v[i, pl.ds(c * 16, 16)] = accs[c * 2] + accs[c * 2 + 1]

    start(0, rows_a, sem_a)

    def body(j, carry):
        i = j * 2
        start(i + 1, rows_b, sem_b)
        wait(i, rows_a, sem_a)
        accum_bag(i, rows_a)

        @pl.when(i + 2 < BPW)
        def _():
            start(i + 2, rows_a, sem_a)

        wait(i + 1, rows_b, sem_b)
        accum_bag(i + 1, rows_b)
        return carry

    lax.fori_loop(0, BPW // 2, body, 0)

    pltpu.sync_copy(out_v, out_hbm.at[pl.ds(base, BPW), :])


def _sc_bag_sum(data, tbl):
    mesh = plsc.VectorSubcoreMesh(core_axis_name="c", subcore_axis_name="s",
                                  num_cores=NC, num_subcores=NS)
    return pl.kernel(
        _sc_body,
        out_type=jax.ShapeDtypeStruct((B, EMB), jnp.float32),
        mesh=mesh,
        compiler_params=pltpu.CompilerParams(use_tc_tiling_on_sc=False),
        scratch_types=[
            pltpu.VMEM((BPW, L), jnp.int32),
            pltpu.VMEM((BPW, L), jnp.int32),
            pltpu.VMEM((L, EMB), jnp.float32),
            pltpu.VMEM((L, EMB), jnp.float32),
            pltpu.VMEM((BPW, EMB), jnp.float32),
            pltpu.SemaphoreType.DMA,
            pltpu.SemaphoreType.DMA,
        ],
    )(data, tbl)


def _tc_body(sums_ref, data_ref, len_ref, t0_ref, wp_ref, bp_ref, out_ref):
    n0 = jnp.sum((data_ref[...] == 0).astype(jnp.float32), axis=1,
                 keepdims=True)
    pooled = (sums_ref[...] - n0 * t0_ref[...]) / len_ref[...].astype(
        jnp.float32)
    logits = jnp.dot(pooled, wp_ref[...],
                     preferred_element_type=jnp.float32) + bp_ref[...]
    m = jnp.max(logits, axis=-1, keepdims=True)
    e = jnp.exp(logits - m)
    s = jnp.sum(e, axis=-1, keepdims=True)
    out_full = logits - m - jnp.log(s)
    out_ref[...] = out_full[:, :NCLASS]


def kernel(data, length, table, W, b):
    data = data.astype(jnp.int32)
    tbl = _transpose_pack(table.T).reshape(2 * VOCAB, EMB)
    sums = _sc_bag_sum(data, tbl)

    wp = jnp.zeros((EMB, LANE_PAD), jnp.float32).at[:, :NCLASS].set(W.T)
    bp = jnp.full((1, LANE_PAD), -1e30, jnp.float32).at[0, :NCLASS].set(b)
    t0 = tbl[0:1, :]
    len2 = length.astype(jnp.int32).reshape(B, 1)

    out = pl.pallas_call(
        _tc_body,
        out_shape=jax.ShapeDtypeStruct((B, NCLASS), jnp.float32),
    )(sums, data, len2, t0, wp, bp)
    return out


# speedup vs baseline: 4.4151x; 1.0461x over previous
import jax
import jax.numpy as jnp
from jax import lax
from jax.experimental import pallas as pl
from jax.experimental.pallas import tpu as pltpu
from jax.experimental.pallas import tpu_sc as plsc

B = 4096
L = 200
EMB = 64
VOCAB = 1000000
NCLASS = 5
LANE_PAD = 128

NC = 2
NS = 16
NW = NC * NS
BPW = B // NW

CB = 16384

SPLIT0 = 128
SPLIT1 = L - SPLIT0


def _tp_body(in_ref, out_ref):
    x = in_ref[...]
    xt = jnp.transpose(x)
    z = jnp.zeros((CB, EMB), jnp.float32)
    out_ref[...] = jnp.concatenate([xt, z], axis=1)


def _transpose_pack(table_t):
    return pl.pallas_call(
        _tp_body,
        grid=((VOCAB + CB - 1) // CB,),
        in_specs=[pl.BlockSpec((EMB, CB), lambda j: (0, j))],
        out_specs=pl.BlockSpec((CB, 2 * EMB), lambda j: (j, 0)),
        out_shape=jax.ShapeDtypeStruct((VOCAB, 2 * EMB), jnp.float32),
    )(table_t)


def _sc_body(data_hbm, tbl_hbm, out_hbm, idx_v, idx2_v, rows_a, rows_b,
             out_v, sem_a, sem_b):
    wid = lax.axis_index("s") * NC + lax.axis_index("c")
    base = wid * BPW

    pltpu.sync_copy(data_hbm.at[pl.ds(base, BPW), :], idx_v)

    def remap_body(i, carry):
        for c in range(12):
            o = c * 16
            idx2_v[i, pl.ds(o, 16)] = idx_v[i, pl.ds(o, 16)] * 2
        idx2_v[i, pl.ds(L - 16, 16)] = idx_v[i, pl.ds(L - 16, 16)] * 2
        return carry

    lax.fori_loop(0, BPW, remap_body, 0)

    def start(i, rows, sem):
        pltpu.async_copy(tbl_hbm.at[idx2_v.at[i, pl.ds(0, SPLIT0)]],
                         rows.at[pl.ds(0, SPLIT0), :], sem)
        pltpu.async_copy(tbl_hbm.at[idx2_v.at[i, pl.ds(SPLIT0, SPLIT1)]],
                         rows.at[pl.ds(SPLIT0, SPLIT1), :], sem)

    def wait(i, rows, sem):
        pltpu.make_async_copy(tbl_hbm.at[idx2_v.at[i, pl.ds(0, SPLIT0)]],
                              rows.at[pl.ds(0, SPLIT0), :], sem).wait()
        pltpu.make_async_copy(tbl_hbm.at[idx2_v.at[i, pl.ds(SPLIT0, SPLIT1)]],
                              rows.at[pl.ds(SPLIT0, SPLIT1), :], sem).wait()

    def accum_bag(i, rows):
        def rbody(r, accs):
            accs = list(accs)
            rb = r * 8
            for u in range(8):
                for c in range(4):
                    v = rows[rb + u, pl.ds(c * 16, 16)]
                    k = c * 2 + (u & 1)
                    accs[k] = accs[k] + v
            return tuple(accs)

        z = jnp.zeros((16,), jnp.float32)
        accs = lax.fori_loop(0, L // 8, rbody, (z,) * 8)
        for c in range(4):
            out_v[i, pl.ds(c * 16, 16)] = accs[c * 2] + accs[c * 2 + 1]

    start(0, rows_a, sem_a)

    def body(j, carry):
        i = j * 2
        start(i + 1, rows_b, sem_b)
        wait(i, rows_a, sem_a)
        accum_bag(i, rows_a)

        @pl.when(i + 2 < BPW)
        def _():
            start(i + 2, rows_a, sem_a)

        wait(i + 1, rows_b, sem_b)
        accum_bag(i + 1, rows_b)
        return carry

    lax.fori_loop(0, BPW // 2, body, 0)

    pltpu.sync_copy(out_v, out_hbm.at[pl.ds(base, BPW), :])


def _sc_bag_sum(data, tbl):
    mesh = plsc.VectorSubcoreMesh(core_axis_name="c", subcore_axis_name="s",
                                  num_cores=NC, num_subcores=NS)
    return pl.kernel(
        _sc_body,
        out_type=jax.ShapeDtypeStruct((B, EMB), jnp.float32),
        mesh=mesh,
        compiler_params=pltpu.CompilerParams(use_tc_tiling_on_sc=False),
        scratch_types=[
            pltpu.VMEM((BPW, L), jnp.int32),
            pltpu.VMEM((BPW, L), jnp.int32),
            pltpu.VMEM((L, EMB), jnp.float32),
            pltpu.VMEM((L, EMB), jnp.float32),
            pltpu.VMEM((BPW, EMB), jnp.float32),
            pltpu.SemaphoreType.DMA,
            pltpu.SemaphoreType.DMA,
        ],
    )(data, tbl)


def _tc_body(sums_ref, data_ref, len_ref, t0_ref, wp_ref, bp_ref, out_ref):
    n0 = jnp.sum((data_ref[...] == 0).astype(jnp.float32), axis=1,
                 keepdims=True)
    pooled = (sums_ref[...] - n0 * t0_ref[...]) / len_ref[...].astype(
        jnp.float32)
    logits = jnp.dot(pooled, wp_ref[...],
                     preferred_element_type=jnp.float32) + bp_ref[...]
    m = jnp.max(logits, axis=-1, keepdims=True)
    e = jnp.exp(logits - m)
    s = jnp.sum(e, axis=-1, keepdims=True)
    out_full = logits - m - jnp.log(s)
    out_ref[...] = out_full[:, :NCLASS]


def kernel(data, length, table, W, b):
    data = data.astype(jnp.int32)
    tbl = _transpose_pack(table.T).reshape(2 * VOCAB, EMB)
    sums = _sc_bag_sum(data, tbl)

    wp = jnp.zeros((EMB, LANE_PAD), jnp.float32).at[:, :NCLASS].set(W.T)
    bp = jnp.full((1, LANE_PAD), -1e30, jnp.float32).at[0, :NCLASS].set(b)
    t0 = tbl[0:1, :]
    len2 = length.astype(jnp.int32).reshape(B, 1)

    out = pl.pallas_call(
        _tc_body,
        out_shape=jax.ShapeDtypeStruct((B, NCLASS), jnp.float32),
    )(sums, data, len2, t0, wp, bp)
    return out


# speedup vs baseline: 4.4948x; 1.0181x over previous
import jax
import jax.numpy as jnp
from jax import lax
from jax.experimental import pallas as pl
from jax.experimental.pallas import tpu as pltpu
from jax.experimental.pallas import tpu_sc as plsc

B = 4096
L = 200
EMB = 64
VOCAB = 1000000
NCLASS = 5
LANE_PAD = 128

NC = 2
NS = 16
NW = NC * NS
BPW = B // NW

CB = 32768

SPLIT0 = 128
SPLIT1 = L - SPLIT0


def _tp_body(in_ref, out_ref):
    x = in_ref[...]
    xt = jnp.transpose(x)
    z = jnp.zeros((CB, EMB), jnp.float32)
    out_ref[...] = jnp.concatenate([xt, z], axis=1)


def _transpose_pack(table_t):
    return pl.pallas_call(
        _tp_body,
        grid=((VOCAB + CB - 1) // CB,),
        in_specs=[pl.BlockSpec((EMB, CB), lambda j: (0, j))],
        out_specs=pl.BlockSpec((CB, 2 * EMB), lambda j: (j, 0)),
        out_shape=jax.ShapeDtypeStruct((VOCAB, 2 * EMB), jnp.float32),
    )(table_t)


def _sc_body(data_hbm, tbl_hbm, out_hbm, idx_v, idx2_v, rows_a, rows_b,
             out_v, sem_a, sem_b):
    wid = lax.axis_index("s") * NC + lax.axis_index("c")
    base = wid * BPW

    pltpu.sync_copy(data_hbm.at[pl.ds(base, BPW), :], idx_v)

    def remap_body(i, carry):
        for c in range(12):
            o = c * 16
            idx2_v[i, pl.ds(o, 16)] = idx_v[i, pl.ds(o, 16)] * 2
        idx2_v[i, pl.ds(L - 16, 16)] = idx_v[i, pl.ds(L - 16, 16)] * 2
        return carry

    lax.fori_loop(0, BPW, remap_body, 0)

    def start(i, rows, sem):
        pltpu.async_copy(tbl_hbm.at[idx2_v.at[i, pl.ds(0, SPLIT0)]],
                         rows.at[pl.ds(0, SPLIT0), :], sem)
        pltpu.async_copy(tbl_hbm.at[idx2_v.at[i, pl.ds(SPLIT0, SPLIT1)]],
                         rows.at[pl.ds(SPLIT0, SPLIT1), :], sem)

    def wait(i, rows, sem):
        pltpu.make_async_copy(tbl_hbm.at[idx2_v.at[i, pl.ds(0, SPLIT0)]],
                              rows.at[pl.ds(0, SPLIT0), :], sem).wait()
        pltpu.make_async_copy(tbl_hbm.at[idx2_v.at[i, pl.ds(SPLIT0, SPLIT1)]],
                              rows.at[pl.ds(SPLIT0, SPLIT1), :], sem).wait()

    def accum_bag(i, rows):
        def rbody(r, accs):
            accs = list(accs)
            rb = r * 8
            for u in range(8):
                for c in range(4):
                    v = rows[rb + u, pl.ds(c * 16, 16)]
                    k = c * 2 + (u & 1)
                    accs[k] = accs[k] + v
            return tuple(accs)

        z = jnp.zeros((16,), jnp.float32)
        accs = lax.fori_loop(0, L // 8, rbody, (z,) * 8)
        for c in range(4):
            out_v[i, pl.ds(c * 16, 16)] = accs[c * 2] + accs[c * 2 + 1]

    start(0, rows_a, sem_a)

    def body(j, carry):
        i = j * 2
        start(i + 1, rows_b, sem_b)
        wait(i, rows_a, sem_a)
        accum_bag(i, rows_a)

        @pl.when(i + 2 < BPW)
        def _():
            start(i + 2, rows_a, sem_a)

        wait(i + 1, rows_b, sem_b)
        accum_bag(i + 1, rows_b)
        return carry

    lax.fori_loop(0, BPW // 2, body, 0)

    pltpu.sync_copy(out_v, out_hbm.at[pl.ds(base, BPW), :])


def _sc_bag_sum(data, tbl):
    mesh = plsc.VectorSubcoreMesh(core_axis_name="c", subcore_axis_name="s",
                                  num_cores=NC, num_subcores=NS)
    return pl.kernel(
        _sc_body,
        out_type=jax.ShapeDtypeStruct((B, EMB), jnp.float32),
        mesh=mesh,
        compiler_params=pltpu.CompilerParams(use_tc_tiling_on_sc=False),
        scratch_types=[
            pltpu.VMEM((BPW, L), jnp.int32),
            pltpu.VMEM((BPW, L), jnp.int32),
            pltpu.VMEM((L, EMB), jnp.float32),
            pltpu.VMEM((L, EMB), jnp.float32),
            pltpu.VMEM((BPW, EMB), jnp.float32),
            pltpu.SemaphoreType.DMA,
            pltpu.SemaphoreType.DMA,
        ],
    )(data, tbl)


def _tc_body(sums_ref, data_ref, len_ref, t0_ref, wp_ref, bp_ref, out_ref):
    n0 = jnp.sum((data_ref[...] == 0).astype(jnp.float32), axis=1,
                 keepdims=True)
    pooled = (sums_ref[...] - n0 * t0_ref[...]) / len_ref[...].astype(
        jnp.float32)
    logits = jnp.dot(pooled, wp_ref[...],
                     preferred_element_type=jnp.float32) + bp_ref[...]
    m = jnp.max(logits, axis=-1, keepdims=True)
    e = jnp.exp(logits - m)
    s = jnp.sum(e, axis=-1, keepdims=True)
    out_full = logits - m - jnp.log(s)
    out_ref[...] = out_full[:, :NCLASS]


def kernel(data, length, table, W, b):
    data = data.astype(jnp.int32)
    tbl = _transpose_pack(table.T).reshape(2 * VOCAB, EMB)
    sums = _sc_bag_sum(data, tbl)

    wp = jnp.zeros((EMB, LANE_PAD), jnp.float32).at[:, :NCLASS].set(W.T)
    bp = jnp.full((1, LANE_PAD), -1e30, jnp.float32).at[0, :NCLASS].set(b)
    t0 = tbl[0:1, :]
    len2 = length.astype(jnp.int32).reshape(B, 1)

    out = pl.pallas_call(
        _tc_body,
        out_shape=jax.ShapeDtypeStruct((B, NCLASS), jnp.float32),
    )(sums, data, len2, t0, wp, bp)
    return out


# speedup vs baseline: 5.0176x; 1.1163x over previous
import jax
import jax.numpy as jnp
from jax import lax
from jax.experimental import pallas as pl
from jax.experimental.pallas import tpu as pltpu
from jax.experimental.pallas import tpu_sc as plsc

B = 4096
L = 200
EMB = 64
VOCAB = 1000000
NCLASS = 5
LANE_PAD = 128

NC = 2
NS = 16
NW = NC * NS
BPW = B // NW

CB = 32768

SPLIT0 = 128
SPLIT1 = L - SPLIT0


def _tp_body(in_ref, out_ref):
    x = in_ref[...]
    xt = jnp.transpose(x)
    z = jnp.zeros((CB, EMB), jnp.float32)
    out_ref[...] = jnp.concatenate([xt, z], axis=1)


def _transpose_pack(table_t):
    return pl.pallas_call(
        _tp_body,
        grid=((VOCAB + CB - 1) // CB,),
        in_specs=[pl.BlockSpec((EMB, CB), lambda j: (0, j))],
        out_specs=pl.BlockSpec((CB, 2 * EMB), lambda j: (j, 0)),
        out_shape=jax.ShapeDtypeStruct((VOCAB, 2 * EMB), jnp.float32),
    )(table_t)


def _sc_body(data_hbm, tbl_hbm, out_hbm, idx_v, idx2_v, rows_a, rows_b,
             rows_c, rows_d, out_v, sem_a, sem_b, sem_c, sem_d):
    wid = lax.axis_index("s") * NC + lax.axis_index("c")
    base = wid * BPW

    pltpu.sync_copy(data_hbm.at[pl.ds(base, BPW), :], idx_v)

    def remap_body(i, carry):
        for c in range(12):
            o = c * 16
            idx2_v[i, pl.ds(o, 16)] = idx_v[i, pl.ds(o, 16)] * 2
        idx2_v[i, pl.ds(L - 16, 16)] = idx_v[i, pl.ds(L - 16, 16)] * 2
        return carry

    lax.fori_loop(0, BPW, remap_body, 0)

    def start(i, rows, sem):
        pltpu.async_copy(tbl_hbm.at[idx2_v.at[i, pl.ds(0, SPLIT0)]],
                         rows.at[pl.ds(0, SPLIT0), :], sem)
        pltpu.async_copy(tbl_hbm.at[idx2_v.at[i, pl.ds(SPLIT0, SPLIT1)]],
                         rows.at[pl.ds(SPLIT0, SPLIT1), :], sem)

    def wait(i, rows, sem):
        pltpu.make_async_copy(tbl_hbm.at[idx2_v.at[i, pl.ds(0, SPLIT0)]],
                              rows.at[pl.ds(0, SPLIT0), :], sem).wait()
        pltpu.make_async_copy(tbl_hbm.at[idx2_v.at[i, pl.ds(SPLIT0, SPLIT1)]],
                              rows.at[pl.ds(SPLIT0, SPLIT1), :], sem).wait()

    def accum_bag(i, rows):
        def rbody(r, accs):
            accs = list(accs)
            rb = r * 8
            for u in range(8):
                for c in range(4):
                    v = rows[rb + u, pl.ds(c * 16, 16)]
                    k = c * 2 + (u & 1)
                    accs[k] = accs[k] + v
            return tuple(accs)

        z = jnp.zeros((16,), jnp.float32)
        accs = lax.fori_loop(0, L // 8, rbody, (z,) * 8)
        for c in range(4):
            out_v[i, pl.ds(c * 16, 16)] = accs[c * 2] + accs[c * 2 + 1]

    ring = ((rows_a, sem_a), (rows_b, sem_b), (rows_c, sem_c),
            (rows_d, sem_d))
    for p, (rows, sem) in enumerate(ring):
        start(p, rows, sem)

    def body(j, carry):
        i = j * 4
        for p, (rows, sem) in enumerate(ring):
            wait(i + p, rows, sem)
            accum_bag(i + p, rows)

            @pl.when(i + p + 4 < BPW)
            def _():
                start(i + p + 4, rows, sem)
        return carry

    lax.fori_loop(0, BPW // 4, body, 0)

    pltpu.sync_copy(out_v, out_hbm.at[pl.ds(base, BPW), :])


def _sc_bag_sum(data, tbl):
    mesh = plsc.VectorSubcoreMesh(core_axis_name="c", subcore_axis_name="s",
                                  num_cores=NC, num_subcores=NS)
    return pl.kernel(
        _sc_body,
        out_type=jax.ShapeDtypeStruct((B, EMB), jnp.float32),
        mesh=mesh,
        compiler_params=pltpu.CompilerParams(use_tc_tiling_on_sc=False),
        scratch_types=[
            pltpu.VMEM((BPW, L), jnp.int32),
            pltpu.VMEM((BPW, L), jnp.int32),
            pltpu.VMEM((L, EMB), jnp.float32),
            pltpu.VMEM((L, EMB), jnp.float32),
            pltpu.VMEM((L, EMB), jnp.float32),
            pltpu.VMEM((L, EMB), jnp.float32),
            pltpu.VMEM((BPW, EMB), jnp.float32),
            pltpu.SemaphoreType.DMA,
            pltpu.SemaphoreType.DMA,
            pltpu.SemaphoreType.DMA,
            pltpu.SemaphoreType.DMA,
        ],
    )(data, tbl)


def _tc_body(sums_ref, data_ref, len_ref, t0_ref, wp_ref, bp_ref, out_ref):
    n0 = jnp.sum((data_ref[...] == 0).astype(jnp.float32), axis=1,
                 keepdims=True)
    pooled = (sums_ref[...] - n0 * t0_ref[...]) / len_ref[...].astype(
        jnp.float32)
    logits = jnp.dot(pooled, wp_ref[...],
                     preferred_element_type=jnp.float32) + bp_ref[...]
    m = jnp.max(logits, axis=-1, keepdims=True)
    e = jnp.exp(logits - m)
    s = jnp.sum(e, axis=-1, keepdims=True)
    out_full = logits - m - jnp.log(s)
    out_ref[...] = out_full[:, :NCLASS]


def kernel(data, length, table, W, b):
    data = data.astype(jnp.int32)
    tbl = _transpose_pack(table.T).reshape(2 * VOCAB, EMB)
    sums = _sc_bag_sum(data, tbl)

    wp = jnp.zeros((EMB, LANE_PAD), jnp.float32).at[:, :NCLASS].set(W.T)
    bp = jnp.full((1, LANE_PAD), -1e30, jnp.float32).at[0, :NCLASS].set(b)
    t0 = tbl[0:1, :]
    len2 = length.astype(jnp.int32).reshape(B, 1)

    out = pl.pallas_call(
        _tc_body,
        out_shape=jax.ShapeDtypeStruct((B, NCLASS), jnp.float32),
    )(sums, data, len2, t0, wp, bp)
    return out
